# Initial kernel scaffold; baseline (speedup 1.0000x reference)
import jax
import jax.numpy as jnp
from jax.experimental import pallas as pl


def kernel(x, pos, edge_index, ne_W1, ne_b1, ne_W2, ne_b2, ee_W1, ee_b1, ee_W2, ee_b2, fc0_W1, fc0_b1, fc0_W2, fc0_b2, tp0_W, fc1_W1, fc1_b1, fc1_W2, fc1_b2, tp1_W, sp_W1, sp_b1, sp_W2, sp_b2, sp_W3, sp_b3):
    raise NotImplementedError("write your pallas kernel here")


# speedup vs baseline: 1.0003x; 1.0003x over previous
import jax
import jax.numpy as jnp
import numpy as np
from jax.experimental import pallas as pl

N_S = 16
SH_DIM = 9
DIST_EMB = 32
MAX_RADIUS = 10.0


def _node_mlp_body(x_ref, w1_ref, b1_ref, w2_ref, b2_ref, o_ref):
    h = jnp.maximum(
        jnp.dot(x_ref[...], w1_ref[...], preferred_element_type=jnp.float32)
        + b1_ref[...], 0.0)
    o_ref[...] = jnp.dot(h, w2_ref[...], preferred_element_type=jnp.float32) + b2_ref[...]


def _node_mlp(x, w1, b1, w2, b2):
    n, fi = x.shape
    fo = w2.shape[1]
    blk = 1000
    return pl.pallas_call(
        _node_mlp_body,
        grid=(n // blk,),
        in_specs=[
            pl.BlockSpec((blk, fi), lambda i: (i, 0)),
            pl.BlockSpec((fi, w1.shape[1]), lambda i: (0, 0)),
            pl.BlockSpec((w1.shape[1],), lambda i: (0,)),
            pl.BlockSpec((w1.shape[1], fo), lambda i: (0, 0)),
            pl.BlockSpec((fo,), lambda i: (0,)),
        ],
        out_specs=pl.BlockSpec((blk, fo), lambda i: (i, 0)),
        out_shape=jax.ShapeDtypeStruct((n, fo), jnp.float32),
    )(x, w1, b1, w2, b2)


def _gaussian_smearing(dist):
    mu = jnp.linspace(0.0, MAX_RADIUS, DIST_EMB)
    coeff = -0.5 / (MAX_RADIUS / (DIST_EMB - 1)) ** 2
    d = dist[:, None] - mu[None, :]
    return jnp.exp(coeff * d * d)


def _sph_harm(vec):
    n = jnp.sqrt(jnp.sum(vec * vec, axis=-1, keepdims=True) + 1e-12)
    u = vec / n
    x, y, z = u[:, 0], u[:, 1], u[:, 2]
    s3 = np.sqrt(3.0); s15 = np.sqrt(15.0); s5 = np.sqrt(5.0)
    return jnp.stack([
        jnp.ones_like(x),
        s3 * x, s3 * y, s3 * z,
        s15 * x * y, s15 * y * z, 0.5 * s5 * (3.0 * z * z - 1.0),
        s15 * x * z, 0.5 * s15 * (x * x - y * y),
    ], axis=-1)


def kernel(x, pos, edge_index, ne_W1, ne_b1, ne_W2, ne_b2, ee_W1, ee_b1, ee_W2,
           ee_b2, fc0_W1, fc0_b1, fc0_W2, fc0_b2, tp0_W, fc1_W1, fc1_b1,
           fc1_W2, fc1_b2, tp1_W, sp_W1, sp_b1, sp_W2, sp_b2, sp_W3, sp_b3):
    src = edge_index[0]
    dst = edge_index[1]
    edge_vec = pos[dst] - pos[src]
    dist = jnp.sqrt(jnp.sum(edge_vec * edge_vec, axis=-1) + 1e-12)
    edge_attr = _gaussian_smearing(dist)
    edge_sh = _sph_harm(edge_vec)
    h = _node_mlp(x, ne_W1, ne_b1, ne_W2, ne_b2)
    e = jnp.maximum(edge_attr @ ee_W1 + ee_b1, 0.0) @ ee_W2 + ee_b2
    fcs = [(fc0_W1, fc0_b1, fc0_W2, fc0_b2, tp0_W),
           (fc1_W1, fc1_b1, fc1_W2, fc1_b2, tp1_W)]
    n_nodes = x.shape[0]
    for i in range(2):
        fW1, fb1, fW2, fb2, tpW = fcs[i]
        ea = jnp.concatenate([e, h[dst, :N_S], h[src, :N_S]], axis=-1)
        w = jnp.maximum(ea @ fW1 + fb1, 0.0) @ fW2 + fb2
        hs = h[src]
        outer = (hs[:, :, None] * edge_sh[:, None, :]).reshape(hs.shape[0], -1)
        tp_out = (outer @ tpW) * w
        sums = jax.ops.segment_sum(tp_out, dst, num_segments=n_nodes)
        cnt = jax.ops.segment_sum(jnp.ones((tp_out.shape[0],), dtype=tp_out.dtype),
                                  dst, num_segments=n_nodes)
        out = sums / jnp.clip(cnt, 1.0)[:, None]
        h = jnp.pad(h, ((0, 0), (0, out.shape[-1] - h.shape[-1]))) + out
    hn = h[:, :N_S]
    s = jnp.maximum(hn @ sp_W1 + sp_b1, 0.0)
    s = jnp.maximum(s @ sp_W2 + sp_b2, 0.0)
    scores = s @ sp_W3 + sp_b3
    return scores


# speedup vs baseline: 162.6810x; 162.6375x over previous
import functools

import jax
import jax.numpy as jnp
import numpy as np
from jax import lax
from jax.experimental import pallas as pl
from jax.experimental.pallas import tpu as pltpu
from jax.experimental.pallas import tpu_sc as plsc

N_S = 16
SH_DIM = 9
DIST_EMB = 32
MAX_RADIUS = 10.0

_CHUNK = 128
_NC = 2
_NS = 16
_NW = _NC * _NS


def _node_mlp_body(x_ref, p_ref, w1_ref, b1_ref, w2_ref, b2_ref, o_ref):
    h = jnp.maximum(
        jnp.dot(x_ref[...], w1_ref[...], preferred_element_type=jnp.float32)
        + b1_ref[...], 0.0)
    h = jnp.dot(h, w2_ref[...], preferred_element_type=jnp.float32) + b2_ref[...]
    blk = h.shape[0]
    pad = jnp.zeros((blk, 32 - N_S - 3), jnp.float32)
    o_ref[...] = jnp.concatenate([h, p_ref[...], pad], axis=1)


def _node_mlp(x, pos, w1, b1, w2, b2):
    n, fi = x.shape
    blk = 2000
    return pl.pallas_call(
        _node_mlp_body,
        grid=(n // blk,),
        in_specs=[
            pl.BlockSpec((blk, fi), lambda i: (i, 0)),
            pl.BlockSpec((blk, 3), lambda i: (i, 0)),
            pl.BlockSpec((fi, N_S), lambda i: (0, 0)),
            pl.BlockSpec((N_S,), lambda i: (0,)),
            pl.BlockSpec((N_S, N_S), lambda i: (0, 0)),
            pl.BlockSpec((N_S,), lambda i: (0,)),
        ],
        out_specs=pl.BlockSpec((blk, 32), lambda i: (i, 0)),
        out_shape=jax.ShapeDtypeStruct((n, 32), jnp.float32),
    )(x, pos, w1, b1, w2, b2)


def _edge0_body(gs_ref, gd_ref, eeW1_ref, eeb1_ref, eeW2_ref, eeb2_ref,
                fW1_ref, fb1_ref, fW2_ref, fb2_ref, tpWp_ref,
                tp_ref, aux_ref):
    gs = gs_ref[...]
    gd = gd_ref[...]
    blk = gs.shape[0]
    ev = gd[:, N_S:N_S + 3] - gs[:, N_S:N_S + 3]
    exx = ev[:, 0:1]
    eyy = ev[:, 1:2]
    ezz = ev[:, 2:3]
    dist = jnp.sqrt(exx * exx + eyy * eyy + ezz * ezz + 1e-12)
    mu = lax.broadcasted_iota(jnp.int32, (1, DIST_EMB), 1).astype(jnp.float32) * (
        MAX_RADIUS / (DIST_EMB - 1))
    coeff = -0.5 / (MAX_RADIUS / (DIST_EMB - 1)) ** 2
    dmu = dist - mu
    sm = jnp.exp(coeff * dmu * dmu)
    e = jnp.maximum(
        jnp.dot(sm, eeW1_ref[...], preferred_element_type=jnp.float32)
        + eeb1_ref[...], 0.0)
    e = jnp.dot(e, eeW2_ref[...], preferred_element_type=jnp.float32) + eeb2_ref[...]
    ux = exx / dist
    uy = eyy / dist
    uz = ezz / dist
    s3 = np.sqrt(3.0)
    s15 = np.sqrt(15.0)
    s5 = np.sqrt(5.0)
    sh = [
        jnp.ones((blk, 1), jnp.float32),
        s3 * ux, s3 * uy, s3 * uz,
        s15 * ux * uy, s15 * uy * uz, 0.5 * s5 * (3.0 * uz * uz - 1.0),
        s15 * ux * uz, 0.5 * s15 * (ux * ux - uy * uy),
    ]
    hs = gs[:, 0:N_S]
    hd = gd[:, 0:N_S]
    ea = jnp.concatenate([e, hd, hs], axis=1)
    w = jnp.maximum(
        jnp.dot(ea, fW1_ref[...], preferred_element_type=jnp.float32)
        + fb1_ref[...], 0.0)
    w = jnp.dot(w, fW2_ref[...], preferred_element_type=jnp.float32) + fb2_ref[...]
    outer = jnp.concatenate([hs * sh[k] for k in range(SH_DIM)], axis=1)
    tp = jnp.dot(outer, tpWp_ref[...], preferred_element_type=jnp.float32) * w
    ones_col = jnp.where(
        lax.broadcasted_iota(jnp.int32, (blk, 16), 1) == 0, 1.0, 0.0)
    tp_ref[...] = jnp.concatenate([tp, ones_col], axis=1)
    aux_ref[...] = jnp.concatenate(
        [e] + sh + [jnp.zeros((blk, 32 - N_S - SH_DIM), jnp.float32)], axis=1)


def _edge0(gs, gd, eeW1, eeb1, eeW2, eeb2, fW1, fb1, fW2, fb2, tpWp, out_dim):
    e_total = gs.shape[0]
    blk = 2000
    return pl.pallas_call(
        _edge0_body,
        grid=(e_total // blk,),
        in_specs=[
            pl.BlockSpec((blk, 32), lambda i: (i, 0)),
            pl.BlockSpec((blk, 32), lambda i: (i, 0)),
            pl.BlockSpec(eeW1.shape, lambda i: (0, 0)),
            pl.BlockSpec(eeb1.shape, lambda i: (0,)),
            pl.BlockSpec(eeW2.shape, lambda i: (0, 0)),
            pl.BlockSpec(eeb2.shape, lambda i: (0,)),
            pl.BlockSpec(fW1.shape, lambda i: (0, 0)),
            pl.BlockSpec(fb1.shape, lambda i: (0,)),
            pl.BlockSpec(fW2.shape, lambda i: (0, 0)),
            pl.BlockSpec(fb2.shape, lambda i: (0,)),
            pl.BlockSpec(tpWp.shape, lambda i: (0, 0)),
        ],
        out_specs=[
            pl.BlockSpec((blk, out_dim + 16), lambda i: (i, 0)),
            pl.BlockSpec((blk, 32), lambda i: (i, 0)),
        ],
        out_shape=[
            jax.ShapeDtypeStruct((e_total, out_dim + 16), jnp.float32),
            jax.ShapeDtypeStruct((e_total, 32), jnp.float32),
        ],
    )(gs, gd, eeW1, eeb1, eeW2, eeb2, fW1, fb1, fW2, fb2, tpWp)


def _update_body(p0_ref, p1_ref, h16_ref, h1_ref, h1b_ref, rec_ref):
    s = p0_ref[0] + p1_ref[0]
    cnt = s[:, 64:65]
    rec = 1.0 / jnp.maximum(cnt, 1.0)
    out64 = s[:, 0:64] * rec
    blk = out64.shape[0]
    h16 = h16_ref[...][:, 0:N_S]
    h1 = jnp.concatenate(
        [h16, jnp.zeros((blk, 64 - N_S), jnp.float32)], axis=1) + out64
    h1_ref[...] = h1
    h1b_ref[...] = h1[:, 0:N_S]
    rec_ref[...] = rec


def _update(acc0, t0):
    n = t0.shape[0]
    blk = 2000
    return pl.pallas_call(
        _update_body,
        grid=(n // blk,),
        in_specs=[
            pl.BlockSpec((1, blk, 80), lambda i: (0, i, 0)),
            pl.BlockSpec((1, blk, 80), lambda i: (1, i, 0)),
            pl.BlockSpec((blk, 32), lambda i: (i, 0)),
        ],
        out_specs=[
            pl.BlockSpec((blk, 64), lambda i: (i, 0)),
            pl.BlockSpec((blk, N_S), lambda i: (i, 0)),
            pl.BlockSpec((blk, 1), lambda i: (i, 0)),
        ],
        out_shape=[
            jax.ShapeDtypeStruct((n, 64), jnp.float32),
            jax.ShapeDtypeStruct((n, N_S), jnp.float32),
            jax.ShapeDtypeStruct((n, 1), jnp.float32),
        ],
    )(acc0.reshape(2, n, 80), acc0.reshape(2, n, 80), t0)


def _edge1_body(hs_ref, hd_ref, aux_ref, fW1_ref, fb1_ref, fW2_ref, fb2_ref,
                tpWp_ref, o_ref):
    hs = hs_ref[...]
    aux = aux_ref[...]
    e = aux[:, 0:N_S]
    ea = jnp.concatenate([e, hd_ref[...], hs[:, 0:N_S]], axis=1)
    w = jnp.maximum(
        jnp.dot(ea, fW1_ref[...], preferred_element_type=jnp.float32)
        + fb1_ref[...], 0.0)
    w = jnp.dot(w, fW2_ref[...], preferred_element_type=jnp.float32) + fb2_ref[...]
    outer = jnp.concatenate(
        [hs * aux[:, N_S + k:N_S + k + 1] for k in range(SH_DIM)], axis=1)
    o_ref[...] = jnp.dot(
        outer, tpWp_ref[...], preferred_element_type=jnp.float32) * w


def _edge1(hs, hd, aux, fW1, fb1, fW2, fb2, tpWp):
    e_total = hs.shape[0]
    blk = 2000
    return pl.pallas_call(
        _edge1_body,
        grid=(e_total // blk,),
        in_specs=[
            pl.BlockSpec((blk, 64), lambda i: (i, 0)),
            pl.BlockSpec((blk, N_S), lambda i: (i, 0)),
            pl.BlockSpec((blk, 32), lambda i: (i, 0)),
            pl.BlockSpec(fW1.shape, lambda i: (0, 0)),
            pl.BlockSpec(fb1.shape, lambda i: (0,)),
            pl.BlockSpec(fW2.shape, lambda i: (0, 0)),
            pl.BlockSpec(fb2.shape, lambda i: (0,)),
            pl.BlockSpec(tpWp.shape, lambda i: (0, 0)),
        ],
        out_specs=pl.BlockSpec((blk, N_S), lambda i: (i, 0)),
        out_shape=jax.ShapeDtypeStruct((e_total, N_S), jnp.float32),
    )(hs, hd, aux, fW1, fb1, fW2, fb2, tpWp)


def _final_body(p0_ref, p1_ref, h1b_ref, rec_ref, w1_ref, b1_ref, w2_ref,
                b2_ref, w3_ref, b3_ref, o_ref):
    s16 = (p0_ref[0] + p1_ref[0]) * rec_ref[...]
    hn = h1b_ref[...] + s16
    t = jnp.maximum(
        jnp.dot(hn, w1_ref[...], preferred_element_type=jnp.float32)
        + b1_ref[...], 0.0)
    t = jnp.maximum(
        jnp.dot(t, w2_ref[...], preferred_element_type=jnp.float32)
        + b2_ref[...], 0.0)
    o_ref[...] = jnp.dot(
        t, w3_ref[...], preferred_element_type=jnp.float32) + b3_ref[...]


def _final(acc1, h1b, rec, w1, b1, w2, b2, w3, b3):
    n = h1b.shape[0]
    blk = 2000
    return pl.pallas_call(
        _final_body,
        grid=(n // blk,),
        in_specs=[
            pl.BlockSpec((1, blk, N_S), lambda i: (0, i, 0)),
            pl.BlockSpec((1, blk, N_S), lambda i: (1, i, 0)),
            pl.BlockSpec((blk, N_S), lambda i: (i, 0)),
            pl.BlockSpec((blk, 1), lambda i: (i, 0)),
            pl.BlockSpec(w1.shape, lambda i: (0, 0)),
            pl.BlockSpec(b1.shape, lambda i: (0,)),
            pl.BlockSpec(w2.shape, lambda i: (0, 0)),
            pl.BlockSpec(b2.shape, lambda i: (0,)),
            pl.BlockSpec(w3.shape, lambda i: (0, 0)),
            pl.BlockSpec(b3.shape, lambda i: (0,)),
        ],
        out_specs=pl.BlockSpec((blk, 1), lambda i: (i, 0)),
        out_shape=jax.ShapeDtypeStruct((n, 1), jnp.float32),
    )(acc1.reshape(2, n, N_S), acc1.reshape(2, n, N_S), h1b, rec,
      w1, b1, w2, b2, w3, b3)


def _sc_gather2(ta, ia, tb, ib):
    e_total = ia.shape[0]
    da = ta.shape[1]
    db = tb.shape[1]
    nchunk = e_total // _CHUNK
    per_w = (nchunk + _NW - 1) // _NW
    mesh = plsc.VectorSubcoreMesh(core_axis_name="c", subcore_axis_name="s")

    @functools.partial(
        pl.kernel,
        out_type=(
            jax.ShapeDtypeStruct((e_total, da), jnp.float32),
            jax.ShapeDtypeStruct((e_total, db), jnp.float32),
        ),
        mesh=mesh,
        scratch_types=[
            pltpu.VMEM((_CHUNK,), jnp.int32),
            pltpu.VMEM((_CHUNK, da), jnp.float32),
            pltpu.VMEM((_CHUNK,), jnp.int32),
            pltpu.VMEM((_CHUNK, db), jnp.float32),
        ],
        compiler_params=pltpu.CompilerParams(use_tc_tiling_on_sc=False),
    )
    def k(ta_hbm, ia_hbm, tb_hbm, ib_hbm, oa_hbm, ob_hbm,
          ia_v, ra_v, ib_v, rb_v):
        w = lax.axis_index("s") * _NC + lax.axis_index("c")

        def body(j, carry):
            cid = w * per_w + j

            @pl.when(cid < nchunk)
            def _():
                off = cid * _CHUNK
                pltpu.sync_copy(ia_hbm.at[pl.ds(off, _CHUNK)], ia_v)
                pltpu.sync_copy(ta_hbm.at[ia_v], ra_v)
                pltpu.sync_copy(ra_v, oa_hbm.at[pl.ds(off, _CHUNK)])
                pltpu.sync_copy(ib_hbm.at[pl.ds(off, _CHUNK)], ib_v)
                pltpu.sync_copy(tb_hbm.at[ib_v], rb_v)
                pltpu.sync_copy(rb_v, ob_hbm.at[pl.ds(off, _CHUNK)])

            return carry

        lax.fori_loop(0, per_w, body, 0)

    return k(ta, ia, tb, ib)


def _sc_scatter(rows, idx, n_nodes, zrows):
    e_total, d = rows.shape
    nchunk = e_total // _CHUNK
    per_w = (nchunk + _NW - 1) // _NW
    stripe = n_nodes // _NS
    mesh = plsc.VectorSubcoreMesh(core_axis_name="c", subcore_axis_name="s")

    @functools.partial(
        pl.kernel,
        out_type=jax.ShapeDtypeStruct((2 * n_nodes, d), jnp.float32),
        mesh=mesh,
        scratch_types=[
            pltpu.VMEM((_CHUNK,), jnp.int32),
            pltpu.VMEM((_CHUNK, d), jnp.float32),
            pltpu.VMEM((stripe, d), jnp.float32),
            pltpu.VMEM_SHARED((n_nodes, d), jnp.float32),
        ],
        compiler_params=pltpu.CompilerParams(use_tc_tiling_on_sc=False),
    )
    def k(rows_hbm, idx_hbm, z_hbm, out_hbm, idx_v, rv, zb, acc):
        c = lax.axis_index("c")
        s = lax.axis_index("s")
        w = s * _NC + c
        pltpu.sync_copy(z_hbm, zb)
        pltpu.sync_copy(zb, acc.at[pl.ds(s * stripe, stripe)])
        plsc.subcore_barrier()

        def body(j, carry):
            cid = w * per_w + j

            @pl.when(cid < nchunk)
            def _():
                off = cid * _CHUNK
                pltpu.sync_copy(idx_hbm.at[pl.ds(off, _CHUNK)], idx_v)
                pltpu.sync_copy(rows_hbm.at[pl.ds(off, _CHUNK)], rv)
                pltpu.sync_copy(rv, acc.at[idx_v], add=True)

            return carry

        lax.fori_loop(0, per_w, body, 0)
        plsc.subcore_barrier()
        pltpu.sync_copy(acc.at[pl.ds(s * stripe, stripe)], zb)
        pltpu.sync_copy(zb, out_hbm.at[pl.ds(c * n_nodes + s * stripe, stripe)])

    return k(rows, idx, zrows)


def kernel(x, pos, edge_index, ne_W1, ne_b1, ne_W2, ne_b2, ee_W1, ee_b1,
           ee_W2, ee_b2, fc0_W1, fc0_b1, fc0_W2, fc0_b2, tp0_W, fc1_W1,
           fc1_b1, fc1_W2, fc1_b2, tp1_W, sp_W1, sp_b1, sp_W2, sp_b2,
           sp_W3, sp_b3):
    n = x.shape[0]
    src = edge_index[0]
    dst = edge_index[1]

    d0 = tp0_W.shape[1]
    tp0_Wp = tp0_W.reshape(N_S, SH_DIM, d0).transpose(1, 0, 2).reshape(
        N_S * SH_DIM, d0)
    tp1_Wp = tp1_W.reshape(64, SH_DIM, -1).transpose(1, 0, 2).reshape(
        64 * SH_DIM, -1)[:, :N_S]
    fc1_W2_16 = fc1_W2[:, :N_S]
    fc1_b2_16 = fc1_b2[:N_S]

    t0 = _node_mlp(x, pos, ne_W1, ne_b1, ne_W2, ne_b2)
    gs, gd = _sc_gather2(t0, src, t0, dst)
    tp0e, aux = _edge0(gs, gd, ee_W1, ee_b1, ee_W2, ee_b2,
                       fc0_W1, fc0_b1, fc0_W2, fc0_b2, tp0_Wp, d0)
    z80 = jnp.zeros((n // _NS, 80), jnp.float32)
    acc0 = _sc_scatter(tp0e, dst, n, z80)
    h1, h1b, rec = _update(acc0, t0)
    hs, hd = _sc_gather2(h1, src, h1b, dst)
    tp1e = _edge1(hs, hd, aux, fc1_W1, fc1_b1, fc1_W2_16, fc1_b2_16, tp1_Wp)
    z16 = jnp.zeros((n // _NS, N_S), jnp.float32)
    acc1 = _sc_scatter(tp1e, dst, n, z16)
    return _final(acc1, h1b, rec, sp_W1, sp_b1, sp_W2, sp_b2, sp_W3, sp_b3)
